# baseline (speedup 1.0000x reference)
import functools

import jax
import jax.numpy as jnp
from jax import lax
from jax.experimental import pallas as pl
from jax.experimental.pallas import tpu as pltpu
from jax.experimental.pallas import tpu_sc as plsc

N = 10000
E = 320000
H = 128
L = 3
BN_EPS = 1e-5
INV_STD = 1.0 / (1.0 + BN_EPS) ** 0.5

NC = 2
NS = 16
NW = NC * NS
CH = 128
K = (E + NW * CH - 1) // (NW * CH)
EP = NW * CH * K
KF = 112
KS = 48
CT = NS * (KF + KS)
ACC_R = 10240
RPT = ACC_R // NS

_sc_mesh = plsc.VectorSubcoreMesh(
    core_axis_name="c", subcore_axis_name="s", num_cores=NC, num_subcores=NS)


@functools.partial(
    pl.kernel,
    out_type=jax.ShapeDtypeStruct((NC, ACC_R, H), jnp.float32),
    mesh=_sc_mesh,
    scratch_types=[
        pltpu.VMEM_SHARED((ACC_R, H), jnp.float32),
        pltpu.VMEM((K, CH), jnp.int32),
        pltpu.VMEM((CH, H), jnp.float32),
    ],
)
def _deg_kernel(dst_hbm, zeros_hbm, ones_hbm, out_hbm, acc, dst_v, ones_v):
    c = lax.axis_index("c")
    s = lax.axis_index("s")
    wid = c * NS + s
    pltpu.sync_copy(zeros_hbm, acc.at[pl.ds(s * RPT, RPT)])
    pltpu.sync_copy(dst_hbm.at[wid], dst_v)
    pltpu.sync_copy(ones_hbm, ones_v)
    plsc.subcore_barrier()

    @pl.loop(0, K)
    def _chunk(i):
        pltpu.sync_copy(ones_v, acc.at[dst_v.at[i]], add=True)

    plsc.subcore_barrier()
    pltpu.sync_copy(acc.at[pl.ds(s * RPT, RPT)],
                    out_hbm.at[c, pl.ds(s * RPT, RPT)])


@functools.partial(
    pl.kernel,
    out_type=jax.ShapeDtypeStruct((NC, ACC_R, H), jnp.float32),
    mesh=_sc_mesh,
    scratch_types=[
        pltpu.VMEM_SHARED((ACC_R, H), jnp.float32),
        pltpu.VMEM((KF // 2, CH), jnp.int32),
        pltpu.VMEM((KF // 2, CH), jnp.int32),
        pltpu.VMEM((CH, H), jnp.float32),
        pltpu.VMEM((CH, H), jnp.float32),
        pltpu.SemaphoreType.DMA,
        pltpu.SemaphoreType.DMA,
    ],
)
def _agg_kernel(x_hbm, src_hbm, dst_hbm, zeros_hbm, out_hbm,
                acc, src_v, dst_v, rows0_v, rows1_v, sem0, sem1):
    c = lax.axis_index("c")
    s = lax.axis_index("s")
    wid = c * NS + s
    pltpu.sync_copy(zeros_hbm, acc.at[pl.ds(s * RPT, RPT)])
    plsc.subcore_barrier()

    def _pipe(kh):
        for half in range(2):
            pltpu.sync_copy(src_hbm.at[wid, pl.ds(half * kh, kh)],
                            src_v.at[pl.ds(0, kh)])
            pltpu.sync_copy(dst_hbm.at[wid, pl.ds(half * kh, kh)],
                            dst_v.at[pl.ds(0, kh)])
            pltpu.async_copy(x_hbm.at[src_v.at[0]], rows0_v, sem0)

            @pl.loop(0, kh // 2)
            def _chunk(j):
                i0 = 2 * j
                pltpu.make_async_copy(
                    x_hbm.at[src_v.at[i0]], rows0_v, sem0).wait()
                pltpu.async_copy(x_hbm.at[src_v.at[i0 + 1]], rows1_v, sem1)
                pltpu.sync_copy(rows0_v, acc.at[dst_v.at[i0]], add=True)
                pltpu.make_async_copy(
                    x_hbm.at[src_v.at[i0 + 1]], rows1_v, sem1).wait()

                @pl.when(i0 + 2 < kh)
                def _():
                    pltpu.async_copy(x_hbm.at[src_v.at[i0 + 2]], rows0_v,
                                     sem0)

                pltpu.sync_copy(rows1_v, acc.at[dst_v.at[i0 + 1]], add=True)

    @pl.when(c == 0)
    def _():
        _pipe(KF // 2)

    @pl.when(c == 1)
    def _():
        _pipe(KS // 2)

    plsc.subcore_barrier()
    pltpu.sync_copy(acc.at[pl.ds(s * RPT, RPT)],
                    out_hbm.at[c, pl.ds(s * RPT, RPT)])


_BLK = 2000


def _enc_body(h_ref, w_ref, b_ref, o_ref):
    o_ref[...] = jnp.dot(h_ref[...], w_ref[...],
                         preferred_element_type=jnp.float32) + b_ref[...]


def _encoder(h, W_enc, b_enc):
    return pl.pallas_call(
        _enc_body,
        grid=(N // _BLK,),
        in_specs=[
            pl.BlockSpec((_BLK, H), lambda i: (i, 0)),
            pl.BlockSpec((H, H), lambda i: (0, 0)),
            pl.BlockSpec((1, H), lambda i: (0, 0)),
        ],
        out_specs=pl.BlockSpec((_BLK, H), lambda i: (i, 0)),
        out_shape=jax.ShapeDtypeStruct((N, H), jnp.float32),
    )(h, W_enc, b_enc.reshape(1, H))


def _layer_body(x_ref, ns_ref, dg_ref, wt_ref, wb_ref, b_ref, g_ref, be_ref,
                o_ref):
    deg = jnp.maximum(dg_ref[0, :, 0:1] + dg_ref[1, :, 0:1], 1.0)
    hn = (ns_ref[0] + ns_ref[1]) / deg
    z = (jnp.dot(x_ref[...], wt_ref[...], preferred_element_type=jnp.float32)
         + jnp.dot(hn, wb_ref[...], preferred_element_type=jnp.float32)
         + b_ref[...])
    o_ref[...] = jnp.maximum(z, 0.0) * (g_ref[...] * INV_STD) + be_ref[...]


def _layer(x, nsum, degp, Wt, Wb, b, g, be):
    return pl.pallas_call(
        _layer_body,
        grid=(N // _BLK,),
        in_specs=[
            pl.BlockSpec((_BLK, H), lambda i: (i, 0)),
            pl.BlockSpec((NC, _BLK, H), lambda i: (0, i, 0)),
            pl.BlockSpec((NC, _BLK, 16), lambda i: (0, i, 0)),
            pl.BlockSpec((H, H), lambda i: (0, 0)),
            pl.BlockSpec((H, H), lambda i: (0, 0)),
            pl.BlockSpec((1, H), lambda i: (0, 0)),
            pl.BlockSpec((1, H), lambda i: (0, 0)),
            pl.BlockSpec((1, H), lambda i: (0, 0)),
        ],
        out_specs=pl.BlockSpec((_BLK, H), lambda i: (i, 0)),
        out_shape=jax.ShapeDtypeStruct((N, H), jnp.float32),
    )(x, nsum, degp, Wt, Wb, b.reshape(1, H), g.reshape(1, H),
      be.reshape(1, H))


def _readout_body(x_ref, w1_ref, b1_ref, w2_ref, b2_ref, w3_ref, b3_ref,
                  o_ref, acc_ref):
    i = pl.program_id(0)

    @pl.when(i == 0)
    def _():
        acc_ref[...] = jnp.zeros_like(acc_ref)

    acc_ref[...] += jnp.sum(x_ref[...], axis=0, keepdims=True)

    @pl.when(i == pl.num_programs(0) - 1)
    def _():
        hg = acc_ref[...] * (1.0 / N)
        o1 = jnp.maximum(jnp.dot(hg, w1_ref[...],
                                 preferred_element_type=jnp.float32)
                         + b1_ref[...], 0.0)
        o2 = jnp.maximum(jnp.dot(o1, w2_ref[...],
                                 preferred_element_type=jnp.float32)
                         + b2_ref[...], 0.0)
        o_ref[...] = jnp.dot(o2, w3_ref[...],
                             preferred_element_type=jnp.float32) + b3_ref[...]


def _readout(x, W1, b1, W2, b2, W3, b3):
    nc = W3.shape[1]
    return pl.pallas_call(
        _readout_body,
        grid=(N // _BLK,),
        in_specs=[
            pl.BlockSpec((_BLK, H), lambda i: (i, 0)),
            pl.BlockSpec(W1.shape, lambda i: (0, 0)),
            pl.BlockSpec((1, W1.shape[1]), lambda i: (0, 0)),
            pl.BlockSpec(W2.shape, lambda i: (0, 0)),
            pl.BlockSpec((1, W2.shape[1]), lambda i: (0, 0)),
            pl.BlockSpec(W3.shape, lambda i: (0, 0)),
            pl.BlockSpec((1, nc), lambda i: (0, 0)),
        ],
        out_specs=pl.BlockSpec((1, nc), lambda i: (0, 0)),
        out_shape=jax.ShapeDtypeStruct((1, nc), jnp.float32),
        scratch_shapes=[pltpu.VMEM((1, H), jnp.float32)],
    )(x, W1, b1.reshape(1, -1), W2, b2.reshape(1, -1), W3, b3.reshape(1, -1))


def kernel(h, edge_index, e, W_enc, b_enc, Wl, bl, gamma, beta,
           W1, b1, W2, b2, W3, b3):
    src = edge_index[0]
    dst = edge_index[1]
    pad_u = EP - E
    dst3 = jnp.concatenate(
        [dst, N + (jnp.arange(pad_u, dtype=jnp.int32) % (ACC_R - N))]
    ).reshape(NW, K, CH)

    pad_a = CT * CH - E
    src_f = jnp.concatenate([src, jnp.zeros((pad_a,), jnp.int32)])
    dst_f = jnp.concatenate(
        [dst, N + (jnp.arange(pad_a, dtype=jnp.int32) % (ACC_R - N))])

    def _asym(flat):
        chunks = flat.reshape(CT, CH)
        fast = chunks[:NS * KF].reshape(NS, KF, CH)
        slow = chunks[NS * KF:].reshape(NS, KS, CH)
        slow = jnp.pad(slow, ((0, 0), (0, KF - KS), (0, 0)))
        return jnp.concatenate([fast, slow], axis=0)

    srcA = _asym(src_f)
    dstA = _asym(dst_f)

    zeros_h = jnp.zeros((RPT, H), jnp.float32)
    ones_h = jnp.ones((CH, H), jnp.float32)

    degp = _deg_kernel(dst3, zeros_h, ones_h)[:, :, :16]
    x = _encoder(h, W_enc, b_enc)
    for l in range(L):
        nsum = _agg_kernel(x, srcA, dstA, zeros_h)
        x = _layer(x, nsum, degp, Wl[l, :H], Wl[l, H:], bl[l],
                   gamma[l], beta[l])
    return _readout(x, W1, b1, W2, b2, W3, b3)


# speedup vs baseline: 1.8102x; 1.8102x over previous
import functools

import jax
import jax.numpy as jnp
from jax import lax
from jax.experimental import pallas as pl
from jax.experimental.pallas import tpu as pltpu
from jax.experimental.pallas import tpu_sc as plsc

N = 10000
E = 320000
H = 128
L = 3
BN_EPS = 1e-5
INV_STD = 1.0 / (1.0 + BN_EPS) ** 0.5

NC = 2
NS = 16
NW = NC * NS
CH = 128
K = (E + NW * CH - 1) // (NW * CH)
EP = NW * CH * K
KF = 104
KS = 53
CT = NS * (KF + KS)
ACC_R = 10240
RPT = ACC_R // NS

_sc_mesh = plsc.VectorSubcoreMesh(
    core_axis_name="c", subcore_axis_name="s", num_cores=NC, num_subcores=NS)


@functools.partial(
    pl.kernel,
    out_type=jax.ShapeDtypeStruct((NC, ACC_R, H), jnp.float32),
    mesh=_sc_mesh,
    scratch_types=[
        pltpu.VMEM_SHARED((ACC_R, H), jnp.float32),
        pltpu.VMEM((K, CH), jnp.int32),
        pltpu.VMEM((CH, H), jnp.float32),
    ],
)
def _deg_kernel(dst_hbm, zeros_hbm, ones_hbm, out_hbm, acc, dst_v, ones_v):
    c = lax.axis_index("c")
    s = lax.axis_index("s")
    wid = c * NS + s
    pltpu.sync_copy(zeros_hbm, acc.at[pl.ds(s * RPT, RPT)])
    pltpu.sync_copy(dst_hbm.at[wid], dst_v)
    pltpu.sync_copy(ones_hbm, ones_v)
    plsc.subcore_barrier()

    @pl.loop(0, K)
    def _chunk(i):
        pltpu.sync_copy(ones_v, acc.at[dst_v.at[i]], add=True)

    plsc.subcore_barrier()
    pltpu.sync_copy(acc.at[pl.ds(s * RPT, RPT)],
                    out_hbm.at[c, pl.ds(s * RPT, RPT)])


@functools.partial(
    pl.kernel,
    out_type=jax.ShapeDtypeStruct((NC, ACC_R, H), jnp.float32),
    mesh=_sc_mesh,
    scratch_types=[
        pltpu.VMEM_SHARED((ACC_R, H), jnp.float32),
        pltpu.VMEM((KF, CH), jnp.int32),
        pltpu.VMEM((KF, CH), jnp.int32),
        pltpu.VMEM((CH, H), jnp.float32),
        pltpu.SemaphoreType.DMA,
    ],
)
def _agg_kernel(x_hbm, src_hbm, dst_hbm, zeros_hbm, out_hbm,
                acc, src_v, dst_v, rows_v, sem):
    c = lax.axis_index("c")
    s = lax.axis_index("s")
    wid = c * NS + s
    pltpu.sync_copy(zeros_hbm, acc.at[pl.ds(s * RPT, RPT)])
    pltpu.sync_copy(src_hbm.at[wid], src_v)
    pltpu.sync_copy(dst_hbm.at[wid], dst_v)
    plsc.subcore_barrier()

    def _chunk(i):
        pltpu.async_copy(x_hbm.at[src_v.at[i]], rows_v, sem).wait()
        pltpu.sync_copy(rows_v, acc.at[dst_v.at[i]], add=True)

    @pl.when(c == 0)
    def _():
        pl.loop(0, KF)(_chunk)

    @pl.when(c == 1)
    def _():
        pl.loop(0, KS)(_chunk)

    plsc.subcore_barrier()
    pltpu.sync_copy(acc.at[pl.ds(s * RPT, RPT)],
                    out_hbm.at[c, pl.ds(s * RPT, RPT)])


_BLK = 2000


def _enc_body(h_ref, w_ref, b_ref, o_ref):
    o_ref[...] = jnp.dot(h_ref[...], w_ref[...],
                         preferred_element_type=jnp.float32) + b_ref[...]


def _encoder(h, W_enc, b_enc):
    return pl.pallas_call(
        _enc_body,
        grid=(N // _BLK,),
        in_specs=[
            pl.BlockSpec((_BLK, H), lambda i: (i, 0)),
            pl.BlockSpec((H, H), lambda i: (0, 0)),
            pl.BlockSpec((1, H), lambda i: (0, 0)),
        ],
        out_specs=pl.BlockSpec((_BLK, H), lambda i: (i, 0)),
        out_shape=jax.ShapeDtypeStruct((N, H), jnp.float32),
    )(h, W_enc, b_enc.reshape(1, H))


def _layer_body(x_ref, ns_ref, dg_ref, wt_ref, wb_ref, b_ref, g_ref, be_ref,
                o_ref):
    deg = jnp.maximum(dg_ref[0, :, 0:1] + dg_ref[1, :, 0:1], 1.0)
    hn = (ns_ref[0] + ns_ref[1]) / deg
    z = (jnp.dot(x_ref[...], wt_ref[...], preferred_element_type=jnp.float32)
         + jnp.dot(hn, wb_ref[...], preferred_element_type=jnp.float32)
         + b_ref[...])
    o_ref[...] = jnp.maximum(z, 0.0) * (g_ref[...] * INV_STD) + be_ref[...]


def _layer(x, nsum, degp, Wt, Wb, b, g, be):
    return pl.pallas_call(
        _layer_body,
        grid=(N // _BLK,),
        in_specs=[
            pl.BlockSpec((_BLK, H), lambda i: (i, 0)),
            pl.BlockSpec((NC, _BLK, H), lambda i: (0, i, 0)),
            pl.BlockSpec((NC, _BLK, 16), lambda i: (0, i, 0)),
            pl.BlockSpec((H, H), lambda i: (0, 0)),
            pl.BlockSpec((H, H), lambda i: (0, 0)),
            pl.BlockSpec((1, H), lambda i: (0, 0)),
            pl.BlockSpec((1, H), lambda i: (0, 0)),
            pl.BlockSpec((1, H), lambda i: (0, 0)),
        ],
        out_specs=pl.BlockSpec((_BLK, H), lambda i: (i, 0)),
        out_shape=jax.ShapeDtypeStruct((N, H), jnp.float32),
    )(x, nsum, degp, Wt, Wb, b.reshape(1, H), g.reshape(1, H),
      be.reshape(1, H))


def _readout_body(x_ref, w1_ref, b1_ref, w2_ref, b2_ref, w3_ref, b3_ref,
                  o_ref, acc_ref):
    i = pl.program_id(0)

    @pl.when(i == 0)
    def _():
        acc_ref[...] = jnp.zeros_like(acc_ref)

    acc_ref[...] += jnp.sum(x_ref[...], axis=0, keepdims=True)

    @pl.when(i == pl.num_programs(0) - 1)
    def _():
        hg = acc_ref[...] * (1.0 / N)
        o1 = jnp.maximum(jnp.dot(hg, w1_ref[...],
                                 preferred_element_type=jnp.float32)
                         + b1_ref[...], 0.0)
        o2 = jnp.maximum(jnp.dot(o1, w2_ref[...],
                                 preferred_element_type=jnp.float32)
                         + b2_ref[...], 0.0)
        o_ref[...] = jnp.dot(o2, w3_ref[...],
                             preferred_element_type=jnp.float32) + b3_ref[...]


def _readout(x, W1, b1, W2, b2, W3, b3):
    nc = W3.shape[1]
    return pl.pallas_call(
        _readout_body,
        grid=(N // _BLK,),
        in_specs=[
            pl.BlockSpec((_BLK, H), lambda i: (i, 0)),
            pl.BlockSpec(W1.shape, lambda i: (0, 0)),
            pl.BlockSpec((1, W1.shape[1]), lambda i: (0, 0)),
            pl.BlockSpec(W2.shape, lambda i: (0, 0)),
            pl.BlockSpec((1, W2.shape[1]), lambda i: (0, 0)),
            pl.BlockSpec(W3.shape, lambda i: (0, 0)),
            pl.BlockSpec((1, nc), lambda i: (0, 0)),
        ],
        out_specs=pl.BlockSpec((1, nc), lambda i: (0, 0)),
        out_shape=jax.ShapeDtypeStruct((1, nc), jnp.float32),
        scratch_shapes=[pltpu.VMEM((1, H), jnp.float32)],
    )(x, W1, b1.reshape(1, -1), W2, b2.reshape(1, -1), W3, b3.reshape(1, -1))


def kernel(h, edge_index, e, W_enc, b_enc, Wl, bl, gamma, beta,
           W1, b1, W2, b2, W3, b3):
    src = edge_index[0]
    dst = edge_index[1]
    pad_u = EP - E
    dst3 = jnp.concatenate(
        [dst, N + (jnp.arange(pad_u, dtype=jnp.int32) % (ACC_R - N))]
    ).reshape(NW, K, CH)

    pad_a = CT * CH - E
    src_f = jnp.concatenate([src, jnp.zeros((pad_a,), jnp.int32)])
    dst_f = jnp.concatenate(
        [dst, N + (jnp.arange(pad_a, dtype=jnp.int32) % (ACC_R - N))])

    def _asym(flat):
        chunks = flat.reshape(CT, CH)
        fast = chunks[:NS * KF].reshape(NS, KF, CH)
        slow = chunks[NS * KF:].reshape(NS, KS, CH)
        slow = jnp.pad(slow, ((0, 0), (0, KF - KS), (0, 0)))
        return jnp.concatenate([fast, slow], axis=0)

    srcA = _asym(src_f)
    dstA = _asym(dst_f)

    zeros_h = jnp.zeros((RPT, H), jnp.float32)
    ones_h = jnp.ones((CH, H), jnp.float32)

    degp = _deg_kernel(dst3, zeros_h, ones_h)[:, :, :16]
    x = _encoder(h, W_enc, b_enc)
    for l in range(L):
        nsum = _agg_kernel(x, srcA, dstA, zeros_h)
        x = _layer(x, nsum, degp, Wl[l, :H], Wl[l, H:], bl[l],
                   gamma[l], beta[l])
    return _readout(x, W1, b1, W2, b2, W3, b3)
